# baseline (speedup 1.0000x reference)
import functools

import numpy as np

import jax
import jax.numpy as jnp
from jax import lax
from jax.experimental import pallas as pl
from jax.experimental.pallas import tpu as pltpu
from jax.experimental.pallas import tpu_sc as plsc

N_NODES = 10000
N_EDGES = 160000
D = 256
DH = 128
HID = 512
NGP = 6656
NG = 6640
B = 256
EPS = 1e-5

_ROWS_PER_TILE = 640

_PERM128 = np.zeros(128, np.int32)
for _k in range(4):
    for _j in range(16):
        _PERM128[32 * _k + 2 * _j] = 32 * _k + _j
        _PERM128[32 * _k + 2 * _j + 1] = 32 * _k + 16 + _j
_PERM256 = np.concatenate([_PERM128, 128 + _PERM128])
_K = 128
_EPT = 10240


def _ln(x, g, b):
    m = jnp.mean(x, axis=-1, keepdims=True)
    v = jnp.mean((x - m) ** 2, axis=-1, keepdims=True)
    return (x - m) * lax.rsqrt(v + EPS) * g + b


def _pre0_body(x_ref, g_ref, b_ref, w_ref, hw0_ref, hw1_ref):
    h = _ln(x_ref[...], g_ref[...], b_ref[...])
    hw = jnp.dot(h, w_ref[...], preferred_element_type=jnp.float32)
    hw0_ref[...] = hw[:, :DH].astype(jnp.bfloat16)
    hw1_ref[...] = hw[:, DH:].astype(jnp.bfloat16)


def _tc_pre0(x, g, b, w):
    R = 400
    grid = (N_NODES // R,)
    return pl.pallas_call(
        _pre0_body,
        grid=grid,
        in_specs=[
            pl.BlockSpec((R, D), lambda i: (i, 0)),
            pl.BlockSpec((1, D), lambda i: (0, 0)),
            pl.BlockSpec((1, D), lambda i: (0, 0)),
            pl.BlockSpec((D, D), lambda i: (0, 0)),
        ],
        out_specs=[
            pl.BlockSpec((R, DH), lambda i: (i, 0)),
            pl.BlockSpec((R, DH), lambda i: (i, 0)),
        ],
        out_shape=[
            jax.ShapeDtypeStruct((N_NODES, DH), jnp.bfloat16),
            jax.ShapeDtypeStruct((N_NODES, DH), jnp.bfloat16),
        ],
    )(x, g, b, w)


def _mid_body(a0_ref, a1_ref, bp_ref, xp_ref, g_ref, b_ref, w_ref,
              xn_ref, hw0_ref, hw1_ref):
    agg = jnp.concatenate([a0_ref[...], a1_ref[...]], axis=1)
    xn = jnp.maximum(agg + bp_ref[...], 0.0) + xp_ref[...]
    xn_ref[...] = xn
    h = _ln(xn, g_ref[...], b_ref[...])
    hw = jnp.dot(h, w_ref[...], preferred_element_type=jnp.float32)
    hw0_ref[...] = hw[:, :DH].astype(jnp.bfloat16)
    hw1_ref[...] = hw[:, DH:].astype(jnp.bfloat16)


def _tc_mid(a0, a1, bp, xp, g, b, w):
    R = 400
    grid = (N_NODES // R,)
    return pl.pallas_call(
        _mid_body,
        grid=grid,
        in_specs=[
            pl.BlockSpec((R, DH), lambda i: (i, 0)),
            pl.BlockSpec((R, DH), lambda i: (i, 0)),
            pl.BlockSpec((1, D), lambda i: (0, 0)),
            pl.BlockSpec((R, D), lambda i: (i, 0)),
            pl.BlockSpec((1, D), lambda i: (0, 0)),
            pl.BlockSpec((1, D), lambda i: (0, 0)),
            pl.BlockSpec((D, D), lambda i: (0, 0)),
        ],
        out_specs=[
            pl.BlockSpec((R, D), lambda i: (i, 0)),
            pl.BlockSpec((R, DH), lambda i: (i, 0)),
            pl.BlockSpec((R, DH), lambda i: (i, 0)),
        ],
        out_shape=[
            jax.ShapeDtypeStruct((N_NODES, D), jnp.float32),
            jax.ShapeDtypeStruct((N_NODES, DH), jnp.bfloat16),
            jax.ShapeDtypeStruct((N_NODES, DH), jnp.bfloat16),
        ],
    )(a0, a1, bp, xp, g, b, w)


def _head_in_body(ga_ref, gb_ref, b2_ref, xg_ref, m_ref, oe_ref,
                  pw_ref, pb_ref, iw_ref, ib_ref, h0_ref):
    agg = jnp.concatenate([ga_ref[...], gb_ref[...]], axis=1)
    x3 = jnp.maximum(agg + b2_ref[...], 0.0) + xg_ref[...]
    ad = jnp.dot(x3, pw_ref[...], preferred_element_type=jnp.float32) + pb_ref[...]
    m = m_ref[...]
    pert = ad * (1.0 - m) + oe_ref[...] * m
    h0_ref[...] = (jnp.dot(pert, iw_ref[...], preferred_element_type=jnp.float32)
                   + ib_ref[...])


def _tc_head_in(ga, gb, b2, xg, maskf, oe, pw, pb, iw, ib):
    return pl.pallas_call(
        _head_in_body,
        out_shape=jax.ShapeDtypeStruct((B, HID), jnp.float32),
    )(ga, gb, b2, xg, maskf, oe, pw, pb, iw, ib)


def _blocks_body(h0_ref, g_ref, b_ref, w1_ref, b1_ref, w2_ref, b2_ref, h_ref):
    i = pl.program_id(0)

    @pl.when(i == 0)
    def _():
        h_ref[...] = h0_ref[...]

    h = h_ref[...]
    z = _ln(h, g_ref[0], b_ref[0])
    z = jax.nn.gelu(jnp.dot(z, w1_ref[0], preferred_element_type=jnp.float32)
                    + b1_ref[0])
    z = jnp.dot(z, w2_ref[0], preferred_element_type=jnp.float32) + b2_ref[0]
    h_ref[...] = h + z


def _tc_blocks(h0, g, b, w1, b1, w2, b2):
    return pl.pallas_call(
        _blocks_body,
        grid=(6,),
        in_specs=[
            pl.BlockSpec((B, HID), lambda i: (0, 0)),
            pl.BlockSpec((1, 1, HID), lambda i: (i, 0, 0)),
            pl.BlockSpec((1, 1, HID), lambda i: (i, 0, 0)),
            pl.BlockSpec((1, HID, 4 * HID), lambda i: (i, 0, 0)),
            pl.BlockSpec((1, 1, 4 * HID), lambda i: (i, 0, 0)),
            pl.BlockSpec((1, 4 * HID, HID), lambda i: (i, 0, 0)),
            pl.BlockSpec((1, 1, HID), lambda i: (i, 0, 0)),
        ],
        out_specs=pl.BlockSpec((B, HID), lambda i: (0, 0)),
        out_shape=jax.ShapeDtypeStruct((B, HID), jnp.float32),
    )(h0, g[:, None, :], b[:, None, :], w1, b1[:, None, :], w2,
      b2[:, None, :])


def _proj_body(h_ref, w_ref, b_ref, o_ref):
    o_ref[...] = (jnp.dot(h_ref[...], w_ref[...],
                          preferred_element_type=jnp.float32) + b_ref[...])


def _tc_proj(h, w, b):
    return pl.pallas_call(
        _proj_body,
        out_shape=jax.ShapeDtypeStruct((B, 3 * HID), jnp.float32),
    )(h, w, b)


def _logits_body(p_ref, g_ref, o_ref):
    g = g_ref[...]
    for c in range(3):
        pc = p_ref[:, pl.ds(c * HID, HID)]
        o_ref[:, c, :] = lax.dot_general(
            pc, g, (((1,), (1,)), ((), ())),
            preferred_element_type=jnp.float32)


def _tc_logits(proj, gene):
    RB = B // 4
    return pl.pallas_call(
        _logits_body,
        grid=(4,),
        in_specs=[
            pl.BlockSpec((RB, 3 * HID), lambda i: (i, 0)),
            pl.BlockSpec((NG, HID), lambda i: (0, 0)),
        ],
        out_specs=pl.BlockSpec((RB, 3, NG), lambda i: (i, 0, 0)),
        out_shape=jax.ShapeDtypeStruct((B, 3, NG), jnp.float32),
    )(proj, gene)


def _sc_mp(hwa, hwb, src3, dst3, ew3):
    NCHUNK = _EPT // _K
    DW = DH // 2
    mesh = plsc.VectorSubcoreMesh(core_axis_name="c", subcore_axis_name="s")

    @functools.partial(
        pl.kernel,
        out_type=(
            jax.ShapeDtypeStruct((N_NODES, DH), jnp.float32),
            jax.ShapeDtypeStruct((N_NODES, DH), jnp.float32),
        ),
        mesh=mesh,
        compiler_params=pltpu.CompilerParams(use_tc_tiling_on_sc=False),
        scratch_types=[
            pltpu.VMEM((_K,), jnp.int32),
            pltpu.VMEM((_K,), jnp.int32),
            pltpu.VMEM((4, _K // 4), jnp.int32),
            pltpu.VMEM((4, _K // 4), jnp.int32),
            pltpu.VMEM((_K,), jnp.float32),
            pltpu.VMEM((_K,), jnp.float32),
            pltpu.VMEM((_K, DW), jnp.int32),
            pltpu.VMEM((_K, DW), jnp.int32),
            pltpu.VMEM((_K, DH), jnp.float32),
            pltpu.VMEM((_K, DH), jnp.float32),
            pltpu.VMEM_SHARED((N_NODES, DH), jnp.float32),
            pltpu.SemaphoreType.DMA,
            pltpu.SemaphoreType.DMA,
            pltpu.SemaphoreType.DMA,
            pltpu.SemaphoreType.DMA,
            pltpu.SemaphoreType.DMA,
            pltpu.SemaphoreType.DMA,
        ],
    )
    def k(hwa_hbm, hwb_hbm, src_hbm, dst_hbm, ew_hbm, outa_hbm, outb_hbm,
          srcv0, srcv1, dstv0, dstv1, ewv0, ewv1, rowsi0, rowsi1,
          rowsf0, rowsf1, aggs, gsem0, gsem1, ssem0, ssem1, esem0, esem1):
        c = lax.axis_index("c")
        s = lax.axis_index("s")
        rbase = s * _ROWS_PER_TILE
        nrchunk = jnp.where(s < 15, 5, 3)

        def zrow(r, carry):
            for c8 in range(DH // 16):
                rowsf0[r, pl.ds(c8 * 16, 16)] = jnp.zeros((16,), jnp.float32)
            return carry
        lax.fori_loop(0, _K, zrow, 0)

        def zchunk(t, carry):
            pltpu.sync_copy(rowsf0, aggs.at[pl.ds(rbase + t * _K, _K), :])
            return carry
        lax.fori_loop(0, nrchunk, zchunk, 0)

        @pl.when(s == 15)
        def _():
            pltpu.sync_copy(rowsf0.at[pl.ds(0, 16), :],
                            aggs.at[pl.ds(rbase + 3 * _K, 16), :])
        plsc.subcore_barrier()

        srcs = (srcv0, srcv1)
        dsts = (dstv0, dstv1)
        ews = (ewv0, ewv1)
        rowsi = (rowsi0, rowsi1)
        rowsf = (rowsf0, rowsf1)
        gsems = (gsem0, gsem1)
        ssems = (ssem0, ssem1)
        esems = (esem0, esem1)
        NQ = 4
        QK = _K // NQ

        def do_edges(hw_hbm):
            def idx_start(j, b):
                pltpu.async_copy(src_hbm.at[s, j], srcs[b], esems[b])
                pltpu.async_copy(ew_hbm.at[s, j], ews[b], esems[b])
                pltpu.async_copy(dst_hbm.at[s, j], dsts[b], esems[b])

            def idx_wait(j, b):
                pltpu.make_async_copy(src_hbm.at[s, j], srcs[b],
                                      esems[b]).wait()
                pltpu.make_async_copy(ew_hbm.at[s, j], ews[b],
                                      esems[b]).wait()
                pltpu.make_async_copy(dst_hbm.at[s, j], dsts[b],
                                      esems[b]).wait()

            def gather_start(b):
                pltpu.async_copy(hw_hbm.at[srcs[b]], rowsi[b], gsems[b])

            def gather_wait(b):
                pltpu.make_async_copy(hw_hbm.at[srcs[b]], rowsi[b],
                                      gsems[b]).wait()

            def scatter_fire(b):
                for q in range(NQ):
                    pltpu.async_copy(rowsf[b].at[pl.ds(q * QK, QK), :],
                                     aggs.at[dsts[b].at[q]], ssems[b],
                                     add=True)

            def scatter_drain(b):
                for q in range(NQ):
                    pltpu.make_async_copy(rowsf[b].at[pl.ds(q * QK, QK), :],
                                          aggs.at[dsts[b].at[q]],
                                          ssems[b]).wait()

            def scale(b):
                ri = rowsi[b]
                rf = rowsf[b]
                ewv = ews[b]
                for g in range(_K // 16):
                    wv = ewv[pl.ds(g * 16, 16)]

                    def one(e16, cc):
                        w16 = lax.gather(
                            wv, jnp.full((16, 1), e16, jnp.int32),
                            lax.GatherDimensionNumbers(
                                offset_dims=(), collapsed_slice_dims=(0,),
                                start_index_map=(0,)),
                            (1,),
                            mode=lax.GatherScatterMode.PROMISE_IN_BOUNDS)
                        e = g * 16 + e16
                        for c4 in range(DW // 16):
                            wi = ri[e, pl.ds(c4 * 16, 16)]
                            lo = lax.bitcast_convert_type(
                                wi << 16, jnp.float32)
                            hi = lax.bitcast_convert_type(
                                wi & jnp.int32(-65536), jnp.float32)
                            rf[e, pl.ds(c4 * 32, 16)] = lo * w16
                            rf[e, pl.ds(c4 * 32 + 16, 16)] = hi * w16
                        return cc
                    lax.fori_loop(0, 16, one, 0, unroll=4)

            idx_start(0, 0)
            idx_wait(0, 0)
            gather_start(0)

            def pair(p, carry):
                for b in (0, 1):
                    j = 2 * p + b

                    @pl.when(j < NCHUNK)
                    def _():
                        @pl.when(j + 1 < NCHUNK)
                        def _():
                            idx_start(j + 1, 1 - b)

                        gather_wait(b)
                        scale(b)

                        @pl.when(j + 1 < NCHUNK)
                        def _():
                            idx_wait(j + 1, 1 - b)
                            gather_start(1 - b)

                        scatter_fire(b)
                        scatter_drain(b)
                return carry
            lax.fori_loop(0, (NCHUNK + 1) // 2, pair, 0)

        @pl.when(c == 0)
        def _():
            do_edges(hwa_hbm)

        @pl.when(c == 1)
        def _():
            do_edges(hwb_hbm)

        plsc.subcore_barrier()

        def writeout(out_hbm):
            def wchunk(t, carry):
                r0 = rbase + t * _K
                pltpu.sync_copy(aggs.at[pl.ds(r0, _K), :],
                                out_hbm.at[pl.ds(r0, _K), :])
                return carry
            lax.fori_loop(0, nrchunk, wchunk, 0)

            @pl.when(s == 15)
            def _():
                pltpu.sync_copy(aggs.at[pl.ds(rbase + 3 * _K, 16), :],
                                out_hbm.at[pl.ds(rbase + 3 * _K, 16), :])

        @pl.when(c == 0)
        def _():
            writeout(outa_hbm)

        @pl.when(c == 1)
        def _():
            writeout(outb_hbm)

    return k(hwa, hwb, src3, dst3, ew3)


def _sc_gather(safe, x2, a2a, a2b):
    NW = 32
    RPW = B // NW
    mesh = plsc.VectorSubcoreMesh(core_axis_name="c", subcore_axis_name="s")

    @functools.partial(
        pl.kernel,
        out_type=(
            jax.ShapeDtypeStruct((B, D), jnp.float32),
            jax.ShapeDtypeStruct((B, DH), jnp.float32),
            jax.ShapeDtypeStruct((B, DH), jnp.float32),
        ),
        mesh=mesh,
        scratch_types=[
            pltpu.VMEM((RPW,), jnp.int32),
            pltpu.VMEM((RPW, D), jnp.float32),
            pltpu.VMEM((RPW, DH), jnp.float32),
            pltpu.VMEM((RPW, DH), jnp.float32),
            pltpu.SemaphoreType.DMA,
        ],
    )
    def k(safe_hbm, x2_hbm, aa_hbm, ab_hbm, xg_hbm, ga_hbm, gb_hbm,
          idxv, bufx, bufa, bufb, sem):
        c = lax.axis_index("c")
        s = lax.axis_index("s")
        wid = s * 2 + c
        base = wid * RPW
        pltpu.sync_copy(safe_hbm.at[pl.ds(base, RPW)], idxv)
        pltpu.async_copy(x2_hbm.at[idxv], bufx, sem).wait()
        pltpu.sync_copy(bufx, xg_hbm.at[pl.ds(base, RPW), :])
        pltpu.async_copy(aa_hbm.at[idxv], bufa, sem).wait()
        pltpu.sync_copy(bufa, ga_hbm.at[pl.ds(base, RPW), :])
        pltpu.async_copy(ab_hbm.at[idxv], bufb, sem).wait()
        pltpu.sync_copy(bufb, gb_hbm.at[pl.ds(base, RPW), :])

    return k(safe, x2, a2a, a2b)


def kernel(node_indices, edge_index, edge_weight, partial_emb, ln_g, ln_b,
           gcn_w, gcn_b, post_w, post_b, oov_emb, proj_in_w, proj_in_b,
           blk_ln_g, blk_ln_b, blk_w1, blk_b1, blk_w2, blk_b2,
           proj_out_w, proj_out_b, gene_emb):
    EPAD = 16 * _EPT
    npad = EPAD - N_EDGES
    pad_idx = (jnp.arange(npad, dtype=jnp.int32) % N_NODES)
    src = jnp.concatenate([edge_index[0].astype(jnp.int32), pad_idx])
    dst = jnp.concatenate([edge_index[1].astype(jnp.int32), pad_idx])
    ew = jnp.concatenate([edge_weight, jnp.zeros((npad,), jnp.float32)])
    NCHUNK = _EPT // _K
    src = src.reshape(16, NCHUNK, _K)
    dst = dst.reshape(16, NCHUNK, 4, _K // 4)
    ew = ew.reshape(16, NCHUNK, _K)

    def as_words(hw):
        return lax.bitcast_convert_type(
            hw.reshape(N_NODES, DH // 2, 2), jnp.int32)

    wp = gcn_w[:, :, _PERM256]
    hw0a, hw0b = _tc_pre0(partial_emb, ln_g[0:1], ln_b[0:1], wp[0])
    a0a, a0b = _sc_mp(as_words(hw0a), as_words(hw0b), src, dst, ew)
    x1, hw1a, hw1b = _tc_mid(a0a, a0b, gcn_b[0:1], partial_emb,
                             ln_g[1:2], ln_b[1:2], wp[1])
    a1a, a1b = _sc_mp(as_words(hw1a), as_words(hw1b), src, dst, ew)
    x2, hw2a, hw2b = _tc_mid(a1a, a1b, gcn_b[1:2], x1,
                             ln_g[2:3], ln_b[2:3], wp[2])
    a2a, a2b = _sc_mp(as_words(hw2a), as_words(hw2b), src, dst, ew)

    oov = node_indices == -1
    safe = jnp.where(oov, 0, node_indices).astype(jnp.int32)
    xg, ga, gb = _sc_gather(safe, x2, a2a, a2b)
    maskf = oov.astype(jnp.float32)[:, None]

    h0 = _tc_head_in(ga, gb, gcn_b[2:3], xg, maskf, oov_emb,
                     post_w, post_b[None, :], proj_in_w, proj_in_b[None, :])
    h = _tc_blocks(h0, blk_ln_g, blk_ln_b, blk_w1, blk_b1, blk_w2, blk_b2)
    proj = _tc_proj(h, proj_out_w, proj_out_b[None, :])
    return _tc_logits(proj, gene_emb)


# speedup vs baseline: 2.5490x; 2.5490x over previous
import functools

import jax
import jax.numpy as jnp
from jax import lax
from jax.experimental import pallas as pl
from jax.experimental.pallas import tpu as pltpu
from jax.experimental.pallas import tpu_sc as plsc

N_NODES = 10000
N_EDGES = 160000
D = 256
DH = 128
HID = 512
NGP = 6656
NG = 6640
B = 256
EPS = 1e-5

_ROWS_PER_TILE = 640
_K = 128
_EPT = 10240


def _ln(x, g, b):
    m = jnp.mean(x, axis=-1, keepdims=True)
    v = jnp.mean((x - m) ** 2, axis=-1, keepdims=True)
    return (x - m) * lax.rsqrt(v + EPS) * g + b


def _pre0_body(x_ref, g_ref, b_ref, w_ref, hw0_ref, hw1_ref):
    h = _ln(x_ref[...], g_ref[...], b_ref[...])
    hw = jnp.dot(h, w_ref[...], preferred_element_type=jnp.float32)
    hw0_ref[...] = hw[:, :DH]
    hw1_ref[...] = hw[:, DH:]


def _tc_pre0(x, g, b, w):
    R = 400
    grid = (N_NODES // R,)
    return pl.pallas_call(
        _pre0_body,
        grid=grid,
        in_specs=[
            pl.BlockSpec((R, D), lambda i: (i, 0)),
            pl.BlockSpec((1, D), lambda i: (0, 0)),
            pl.BlockSpec((1, D), lambda i: (0, 0)),
            pl.BlockSpec((D, D), lambda i: (0, 0)),
        ],
        out_specs=[
            pl.BlockSpec((R, DH), lambda i: (i, 0)),
            pl.BlockSpec((R, DH), lambda i: (i, 0)),
        ],
        out_shape=[
            jax.ShapeDtypeStruct((N_NODES, DH), jnp.float32),
            jax.ShapeDtypeStruct((N_NODES, DH), jnp.float32),
        ],
    )(x, g, b, w)


def _mid_body(a0_ref, a1_ref, bp_ref, xp_ref, g_ref, b_ref, w_ref,
              xn_ref, hw0_ref, hw1_ref):
    agg = jnp.concatenate([a0_ref[...], a1_ref[...]], axis=1)
    xn = jnp.maximum(agg + bp_ref[...], 0.0) + xp_ref[...]
    xn_ref[...] = xn
    h = _ln(xn, g_ref[...], b_ref[...])
    hw = jnp.dot(h, w_ref[...], preferred_element_type=jnp.float32)
    hw0_ref[...] = hw[:, :DH]
    hw1_ref[...] = hw[:, DH:]


def _tc_mid(a0, a1, bp, xp, g, b, w):
    R = 400
    grid = (N_NODES // R,)
    return pl.pallas_call(
        _mid_body,
        grid=grid,
        in_specs=[
            pl.BlockSpec((R, DH), lambda i: (i, 0)),
            pl.BlockSpec((R, DH), lambda i: (i, 0)),
            pl.BlockSpec((1, D), lambda i: (0, 0)),
            pl.BlockSpec((R, D), lambda i: (i, 0)),
            pl.BlockSpec((1, D), lambda i: (0, 0)),
            pl.BlockSpec((1, D), lambda i: (0, 0)),
            pl.BlockSpec((D, D), lambda i: (0, 0)),
        ],
        out_specs=[
            pl.BlockSpec((R, D), lambda i: (i, 0)),
            pl.BlockSpec((R, DH), lambda i: (i, 0)),
            pl.BlockSpec((R, DH), lambda i: (i, 0)),
        ],
        out_shape=[
            jax.ShapeDtypeStruct((N_NODES, D), jnp.float32),
            jax.ShapeDtypeStruct((N_NODES, DH), jnp.float32),
            jax.ShapeDtypeStruct((N_NODES, DH), jnp.float32),
        ],
    )(a0, a1, bp, xp, g, b, w)


def _head_in_body(ga_ref, gb_ref, b2_ref, xg_ref, m_ref, oe_ref,
                  pw_ref, pb_ref, iw_ref, ib_ref, h0_ref):
    agg = jnp.concatenate([ga_ref[...], gb_ref[...]], axis=1)
    x3 = jnp.maximum(agg + b2_ref[...], 0.0) + xg_ref[...]
    ad = jnp.dot(x3, pw_ref[...], preferred_element_type=jnp.float32) + pb_ref[...]
    m = m_ref[...]
    pert = ad * (1.0 - m) + oe_ref[...] * m
    h0_ref[...] = (jnp.dot(pert, iw_ref[...], preferred_element_type=jnp.float32)
                   + ib_ref[...])


def _tc_head_in(ga, gb, b2, xg, maskf, oe, pw, pb, iw, ib):
    return pl.pallas_call(
        _head_in_body,
        out_shape=jax.ShapeDtypeStruct((B, HID), jnp.float32),
    )(ga, gb, b2, xg, maskf, oe, pw, pb, iw, ib)


def _blocks_body(h0_ref, g_ref, b_ref, w1_ref, b1_ref, w2_ref, b2_ref, h_ref):
    i = pl.program_id(0)

    @pl.when(i == 0)
    def _():
        h_ref[...] = h0_ref[...]

    h = h_ref[...]
    z = _ln(h, g_ref[0], b_ref[0])
    z = jax.nn.gelu(jnp.dot(z, w1_ref[0], preferred_element_type=jnp.float32)
                    + b1_ref[0])
    z = jnp.dot(z, w2_ref[0], preferred_element_type=jnp.float32) + b2_ref[0]
    h_ref[...] = h + z


def _tc_blocks(h0, g, b, w1, b1, w2, b2):
    return pl.pallas_call(
        _blocks_body,
        grid=(6,),
        in_specs=[
            pl.BlockSpec((B, HID), lambda i: (0, 0)),
            pl.BlockSpec((1, 1, HID), lambda i: (i, 0, 0)),
            pl.BlockSpec((1, 1, HID), lambda i: (i, 0, 0)),
            pl.BlockSpec((1, HID, 4 * HID), lambda i: (i, 0, 0)),
            pl.BlockSpec((1, 1, 4 * HID), lambda i: (i, 0, 0)),
            pl.BlockSpec((1, 4 * HID, HID), lambda i: (i, 0, 0)),
            pl.BlockSpec((1, 1, HID), lambda i: (i, 0, 0)),
        ],
        out_specs=pl.BlockSpec((B, HID), lambda i: (0, 0)),
        out_shape=jax.ShapeDtypeStruct((B, HID), jnp.float32),
    )(h0, g[:, None, :], b[:, None, :], w1, b1[:, None, :], w2,
      b2[:, None, :])


def _proj_body(h_ref, w_ref, b_ref, o_ref):
    o_ref[...] = (jnp.dot(h_ref[...], w_ref[...],
                          preferred_element_type=jnp.float32) + b_ref[...])


def _tc_proj(h, w, b):
    return pl.pallas_call(
        _proj_body,
        out_shape=jax.ShapeDtypeStruct((B, 3 * HID), jnp.float32),
    )(h, w, b)


def _logits_body(p_ref, g_ref, o_ref):
    g = g_ref[...]
    for c in range(3):
        pc = p_ref[:, pl.ds(c * HID, HID)]
        o_ref[:, c, :] = lax.dot_general(
            pc, g, (((1,), (1,)), ((), ())),
            preferred_element_type=jnp.float32)


def _tc_logits(proj, gene):
    RB = B // 4
    return pl.pallas_call(
        _logits_body,
        grid=(4,),
        in_specs=[
            pl.BlockSpec((RB, 3 * HID), lambda i: (i, 0)),
            pl.BlockSpec((NG, HID), lambda i: (0, 0)),
        ],
        out_specs=pl.BlockSpec((RB, 3, NG), lambda i: (i, 0, 0)),
        out_shape=jax.ShapeDtypeStruct((B, 3, NG), jnp.float32),
    )(proj, gene)


def _sc_mp(hwa, hwb, src3, dst3, ew3):
    NCHUNK = _EPT // _K
    mesh = plsc.VectorSubcoreMesh(core_axis_name="c", subcore_axis_name="s")

    @functools.partial(
        pl.kernel,
        out_type=(
            jax.ShapeDtypeStruct((N_NODES, DH), jnp.float32),
            jax.ShapeDtypeStruct((N_NODES, DH), jnp.float32),
        ),
        mesh=mesh,
        scratch_types=[
            pltpu.VMEM((NCHUNK, _K), jnp.int32),
            pltpu.VMEM((4, _K // 4), jnp.int32),
            pltpu.VMEM((4, _K // 4), jnp.int32),
            pltpu.VMEM((_K,), jnp.float32),
            pltpu.VMEM((_K,), jnp.float32),
            pltpu.VMEM((_K, DH), jnp.float32),
            pltpu.VMEM((_K, DH), jnp.float32),
            pltpu.VMEM_SHARED((N_NODES, DH), jnp.float32),
            pltpu.SemaphoreType.DMA,
            pltpu.SemaphoreType.DMA,
            pltpu.SemaphoreType.DMA,
            pltpu.SemaphoreType.DMA,
            pltpu.SemaphoreType.DMA,
            pltpu.SemaphoreType.DMA,
        ],
    )
    def k(hwa_hbm, hwb_hbm, src_hbm, dst_hbm, ew_hbm, outa_hbm, outb_hbm,
          srcall, dstv0, dstv1, ewv0, ewv1, rows0, rows1, aggs,
          gsem0, gsem1, ssem0, ssem1, esem0, esem1):
        c = lax.axis_index("c")
        s = lax.axis_index("s")
        rbase = s * _ROWS_PER_TILE
        nrchunk = jnp.where(s < 15, 5, 3)

        pltpu.sync_copy(src_hbm.at[s], srcall)

        def zrow(r, carry):
            for c8 in range(DH // 16):
                rows0[r, pl.ds(c8 * 16, 16)] = jnp.zeros((16,), jnp.float32)
            return carry
        lax.fori_loop(0, _K, zrow, 0)

        def zchunk(t, carry):
            pltpu.sync_copy(rows0, aggs.at[pl.ds(rbase + t * _K, _K), :])
            return carry
        lax.fori_loop(0, nrchunk, zchunk, 0)

        @pl.when(s == 15)
        def _():
            pltpu.sync_copy(rows0.at[pl.ds(0, 16), :],
                            aggs.at[pl.ds(rbase + 3 * _K, 16), :])
        plsc.subcore_barrier()

        bufs = (rows0, rows1)
        dsts = (dstv0, dstv1)
        ews = (ewv0, ewv1)
        gsems = (gsem0, gsem1)
        ssems = (ssem0, ssem1)
        esems = (esem0, esem1)

        def do_edges(hw_hbm):
            def gather_start(j, b):
                pltpu.async_copy(hw_hbm.at[srcall.at[j]], bufs[b], gsems[b])

            def gather_wait(j, b):
                pltpu.make_async_copy(
                    hw_hbm.at[srcall.at[j]], bufs[b], gsems[b]).wait()

            def ewdst_start(j, b):
                pltpu.async_copy(ew_hbm.at[s, j], ews[b], esems[b])
                pltpu.async_copy(dst_hbm.at[s, j], dsts[b], esems[b])

            def ewdst_wait(j, b):
                pltpu.make_async_copy(ew_hbm.at[s, j], ews[b], esems[b]).wait()
                pltpu.make_async_copy(dst_hbm.at[s, j], dsts[b], esems[b]).wait()

            NQ = 4
            QK = _K // NQ

            def scatter_all(j, b):
                for q in range(NQ):
                    pltpu.async_copy(bufs[b].at[pl.ds(q * QK, QK), :],
                                     aggs.at[dsts[b].at[q]], ssems[b],
                                     add=True)
                for q in range(NQ):
                    pltpu.make_async_copy(bufs[b].at[pl.ds(q * QK, QK), :],
                                          aggs.at[dsts[b].at[q]],
                                          ssems[b]).wait()

            def scale(j, b):
                rows = bufs[b]
                ewv = ews[b]
                for g in range(_K // 16):
                    wv = ewv[pl.ds(g * 16, 16)]

                    def one(e16, cc):
                        w16 = lax.gather(
                            wv, jnp.full((16, 1), e16, jnp.int32),
                            lax.GatherDimensionNumbers(
                                offset_dims=(), collapsed_slice_dims=(0,),
                                start_index_map=(0,)),
                            (1,),
                            mode=lax.GatherScatterMode.PROMISE_IN_BOUNDS)
                        e = g * 16 + e16
                        for c8 in range(DH // 16):
                            sl = pl.ds(c8 * 16, 16)
                            rows[e, sl] = rows[e, sl] * w16
                        return cc
                    lax.fori_loop(0, 16, one, 0, unroll=4)

            ewdst_start(0, 0)
            gather_start(0, 0)

            def pair(p, carry):
                for b in (0, 1):
                    j = 2 * p + b

                    @pl.when(j < NCHUNK)
                    def _():
                        @pl.when(j + 1 < NCHUNK)
                        def _():
                            ewdst_start(j + 1, 1 - b)
                            gather_start(j + 1, 1 - b)

                        gather_wait(j, b)
                        ewdst_wait(j, b)
                        scale(j, b)
                        scatter_all(j, b)
                return carry
            lax.fori_loop(0, (NCHUNK + 1) // 2, pair, 0)

        @pl.when(c == 0)
        def _():
            do_edges(hwa_hbm)

        @pl.when(c == 1)
        def _():
            do_edges(hwb_hbm)

        plsc.subcore_barrier()

        def writeout(out_hbm):
            def wchunk(t, carry):
                r0 = rbase + t * _K
                pltpu.sync_copy(aggs.at[pl.ds(r0, _K), :],
                                out_hbm.at[pl.ds(r0, _K), :])
                return carry
            lax.fori_loop(0, nrchunk, wchunk, 0)

            @pl.when(s == 15)
            def _():
                pltpu.sync_copy(aggs.at[pl.ds(rbase + 3 * _K, 16), :],
                                out_hbm.at[pl.ds(rbase + 3 * _K, 16), :])

        @pl.when(c == 0)
        def _():
            writeout(outa_hbm)

        @pl.when(c == 1)
        def _():
            writeout(outb_hbm)

    return k(hwa, hwb, src3, dst3, ew3)


def _sc_gather(safe, x2, a2a, a2b):
    NW = 32
    RPW = B // NW
    mesh = plsc.VectorSubcoreMesh(core_axis_name="c", subcore_axis_name="s")

    @functools.partial(
        pl.kernel,
        out_type=(
            jax.ShapeDtypeStruct((B, D), jnp.float32),
            jax.ShapeDtypeStruct((B, DH), jnp.float32),
            jax.ShapeDtypeStruct((B, DH), jnp.float32),
        ),
        mesh=mesh,
        scratch_types=[
            pltpu.VMEM((RPW,), jnp.int32),
            pltpu.VMEM((RPW, D), jnp.float32),
            pltpu.VMEM((RPW, DH), jnp.float32),
            pltpu.VMEM((RPW, DH), jnp.float32),
            pltpu.SemaphoreType.DMA,
        ],
    )
    def k(safe_hbm, x2_hbm, aa_hbm, ab_hbm, xg_hbm, ga_hbm, gb_hbm,
          idxv, bufx, bufa, bufb, sem):
        c = lax.axis_index("c")
        s = lax.axis_index("s")
        wid = s * 2 + c
        base = wid * RPW
        pltpu.sync_copy(safe_hbm.at[pl.ds(base, RPW)], idxv)
        pltpu.async_copy(x2_hbm.at[idxv], bufx, sem).wait()
        pltpu.sync_copy(bufx, xg_hbm.at[pl.ds(base, RPW), :])
        pltpu.async_copy(aa_hbm.at[idxv], bufa, sem).wait()
        pltpu.sync_copy(bufa, ga_hbm.at[pl.ds(base, RPW), :])
        pltpu.async_copy(ab_hbm.at[idxv], bufb, sem).wait()
        pltpu.sync_copy(bufb, gb_hbm.at[pl.ds(base, RPW), :])

    return k(safe, x2, a2a, a2b)


def kernel(node_indices, edge_index, edge_weight, partial_emb, ln_g, ln_b,
           gcn_w, gcn_b, post_w, post_b, oov_emb, proj_in_w, proj_in_b,
           blk_ln_g, blk_ln_b, blk_w1, blk_b1, blk_w2, blk_b2,
           proj_out_w, proj_out_b, gene_emb):
    EPAD = 16 * _EPT
    npad = EPAD - N_EDGES
    pad_idx = (jnp.arange(npad, dtype=jnp.int32) % N_NODES)
    src = jnp.concatenate([edge_index[0].astype(jnp.int32), pad_idx])
    dst = jnp.concatenate([edge_index[1].astype(jnp.int32), pad_idx])
    ew = jnp.concatenate([edge_weight, jnp.zeros((npad,), jnp.float32)])
    NCHUNK = _EPT // _K
    src = src.reshape(16, NCHUNK, _K)
    dst = dst.reshape(16, NCHUNK, 4, _K // 4)
    ew = ew.reshape(16, NCHUNK, _K)

    hw0a, hw0b = _tc_pre0(partial_emb, ln_g[0:1], ln_b[0:1], gcn_w[0])
    a0a, a0b = _sc_mp(hw0a, hw0b, src, dst, ew)
    x1, hw1a, hw1b = _tc_mid(a0a, a0b, gcn_b[0:1], partial_emb,
                             ln_g[1:2], ln_b[1:2], gcn_w[1])
    a1a, a1b = _sc_mp(hw1a, hw1b, src, dst, ew)
    x2, hw2a, hw2b = _tc_mid(a1a, a1b, gcn_b[1:2], x1,
                             ln_g[2:3], ln_b[2:3], gcn_w[2])
    a2a, a2b = _sc_mp(hw2a, hw2b, src, dst, ew)

    oov = node_indices == -1
    safe = jnp.where(oov, 0, node_indices).astype(jnp.int32)
    xg, ga, gb = _sc_gather(safe, x2, a2a, a2b)
    maskf = oov.astype(jnp.float32)[:, None]

    h0 = _tc_head_in(ga, gb, gcn_b[2:3], xg, maskf, oov_emb,
                     post_w, post_b[None, :], proj_in_w, proj_in_b[None, :])
    h = _tc_blocks(h0, blk_ln_g, blk_ln_b, blk_w1, blk_b1, blk_w2, blk_b2)
    proj = _tc_proj(h, proj_out_w, proj_out_b[None, :])
    return _tc_logits(proj, gene_emb)
